# baseline (speedup 1.0000x reference)
import functools

import jax
import jax.numpy as jnp
from jax import lax
from jax.experimental import pallas as pl
from jax.experimental.pallas import tpu as pltpu
from jax.experimental.pallas import tpu_sc as plsc


def _gather_kernel(B, S, D):
    mesh = plsc.ScalarSubcoreMesh(axis_name="c", num_cores=1)

    @functools.partial(
        pl.kernel,
        mesh=mesh,
        out_type=jax.ShapeDtypeStruct((B * 4, D), jnp.float32),
        scratch_types=[
            pltpu.SMEM((16,), jnp.int32),
            pltpu.SemaphoreType.DMA,
        ],
    )
    def k(x_hbm, ann_hbm, out_hbm, idx_s, sem):
        pltpu.sync_copy(ann_hbm, idx_s)
        copies = []
        for j in range(B * 4):
            row = idx_s[j] + (j // 4) * S
            copies.append(pltpu.make_async_copy(
                x_hbm.at[pl.ds(row, 1)], out_hbm.at[pl.ds(j, 1)], sem))
        for c in copies:
            c.start()
        for c in copies:
            c.wait()

    return k


def kernel(x, src_tokens, annotation):
    B, S, D = x.shape
    x_flat = x.reshape(B * S, D)
    ann = annotation.reshape(-1).astype(jnp.int32)
    out = _gather_kernel(B, S, D)(x_flat, ann)
    return out.reshape(B, 4 * D)


# speedup vs baseline: 1.0139x; 1.0139x over previous
import functools

import jax
import jax.numpy as jnp
from jax import lax
from jax.experimental import pallas as pl
from jax.experimental.pallas import tpu as pltpu
from jax.experimental.pallas import tpu_sc as plsc


def _gather_kernel(B, S, D):
    mesh = plsc.VectorSubcoreMesh(
        core_axis_name="c", subcore_axis_name="s", num_cores=1)

    @functools.partial(
        pl.kernel,
        mesh=mesh,
        out_type=jax.ShapeDtypeStruct((B * 4, D), jnp.float32),
        scratch_types=[
            pltpu.VMEM((16,), jnp.int32),
            pltpu.VMEM((1, D), jnp.float32),
            pltpu.SemaphoreType.DMA,
        ],
    )
    def k(x_hbm, ann_hbm, out_hbm, idx_v, row_v, sem):
        wid = lax.axis_index("s") + lax.axis_index("c")

        pltpu.sync_copy(ann_hbm, idx_v)
        lane = lax.iota(jnp.int32, 16)
        rows = idx_v[...] + (lane >> 2) * S
        perm = (lane + wid) & 15
        dnums = lax.GatherDimensionNumbers(
            offset_dims=(), collapsed_slice_dims=(0,), start_index_map=(0,))
        idx_v[...] = lax.gather(
            rows, perm.reshape(16, 1), dnums, (1,),
            mode=lax.GatherScatterMode.PROMISE_IN_BOUNDS)
        pltpu.async_copy(x_hbm.at[idx_v.at[pl.ds(0, 1)]], row_v, sem).wait()
        pltpu.sync_copy(row_v, out_hbm.at[pl.ds(wid, 1)])

    return k


def kernel(x, src_tokens, annotation):
    B, S, D = x.shape
    x_flat = x.reshape(B * S, D)
    ann = annotation.reshape(-1).astype(jnp.int32)
    out = _gather_kernel(B, S, D)(x_flat, ann)
    return out.reshape(B, 4 * D)


# speedup vs baseline: 1.0918x; 1.0769x over previous
import functools

import jax
import jax.numpy as jnp
from jax import lax
from jax.experimental import pallas as pl
from jax.experimental.pallas import tpu as pltpu
from jax.experimental.pallas import tpu_sc as plsc


def _gather_kernel(B, S, D):
    mesh = plsc.VectorSubcoreMesh(
        core_axis_name="c", subcore_axis_name="s", num_cores=1)

    @functools.partial(
        pl.kernel,
        mesh=mesh,
        out_type=jax.ShapeDtypeStruct((B, 4 * D), jnp.float32),
        scratch_types=[
            pltpu.VMEM((16,), jnp.int32),
            pltpu.VMEM((1, D), jnp.float32),
            pltpu.SemaphoreType.DMA,
        ],
    )
    def k(x_hbm, ann_hbm, out_hbm, idx_v, row_v, sem):
        wid = lax.axis_index("s") + lax.axis_index("c")

        pltpu.sync_copy(ann_hbm, idx_v)
        lane = lax.iota(jnp.int32, 16)
        b_idx = lane >> 2
        rows = idx_v[...] + b_idx * S
        perm = (lane + wid) & 15
        dnums = lax.GatherDimensionNumbers(
            offset_dims=(), collapsed_slice_dims=(0,), start_index_map=(0,))
        idx_v[...] = lax.gather(
            rows, perm.reshape(16, 1), dnums, (1,),
            mode=lax.GatherScatterMode.PROMISE_IN_BOUNDS)
        pltpu.async_copy(x_hbm.at[idx_v.at[pl.ds(0, 1)]], row_v, sem).wait()
        pltpu.sync_copy(
            row_v, out_hbm.at[pl.ds(wid // 4, 1), pl.ds((wid % 4) * D, D)])

    return k


def kernel(x, src_tokens, annotation):
    B, S, D = x.shape
    x_flat = x.reshape(B * S, D)
    ann = annotation.reshape(-1).astype(jnp.int32)
    return _gather_kernel(B, S, D)(x_flat, ann)
